# Initial kernel scaffold; baseline (speedup 1.0000x reference)
import jax
import jax.numpy as jnp
from jax.experimental import pallas as pl


def kernel(indices, embeddings):
    raise NotImplementedError("write your pallas kernel here")


# speedup vs baseline: 3.2122x; 3.2122x over previous
import functools

import jax
import jax.numpy as jnp
from jax import lax
from jax.experimental import pallas as pl
from jax.experimental.pallas import tpu as pltpu
from jax.experimental.pallas import tpu_sc as plsc

NC = 2
NS = 16
NW = NC * NS
LC_ROWS = 8192


@functools.partial(jax.jit, static_argnums=(3,))
def _lookup(flat_idx, table, lastcol, chunk):
    n = flat_idx.shape[0]
    v, d = table.shape
    n_per_w = n // NW
    n_chunks = n_per_w // chunk
    lc_per_s = LC_ROWS // NS
    mesh = plsc.VectorSubcoreMesh(core_axis_name="c", subcore_axis_name="s")

    @functools.partial(
        pl.kernel,
        mesh=mesh,
        out_type=jax.ShapeDtypeStruct((n, d), jnp.float32),
        compiler_params=pltpu.CompilerParams(needs_layout_passes=False),
        scratch_types=[
            pltpu.VMEM((chunk,), jnp.int32),
            pltpu.VMEM((chunk,), jnp.int32),
            pltpu.VMEM((chunk, d), jnp.float32),
            pltpu.VMEM((chunk, 128), jnp.float32),
            pltpu.VMEM_SHARED((LC_ROWS, 128), jnp.float32),
            pltpu.SemaphoreType.DMA,
            pltpu.SemaphoreType.DMA,
        ],
    )
    def k(idx_hbm, table_hbm, lc_hbm, out_hbm, idx_v, idxhi_v, rows_v,
          slab_v, lc_sh, sem, sem2):
        cid = lax.axis_index("c")
        sid = lax.axis_index("s")
        wid = sid * NC + cid
        base = wid * n_per_w

        so = sid * lc_per_s
        pltpu.sync_copy(
            lc_hbm.at[pl.ds(so, lc_per_s)], lc_sh.at[pl.ds(so, lc_per_s)]
        )
        plsc.subcore_barrier()

        def chunk_body(c, carry):
            off = base + c * chunk
            pltpu.sync_copy(idx_hbm.at[pl.ds(off, chunk)], idx_v)
            for g in range(chunk // 16):
                iv = idx_v[pl.ds(g * 16, 16)]
                idxhi_v[pl.ds(g * 16, 16)] = lax.shift_right_logical(iv, 7)
            main = pltpu.async_copy(
                table_hbm.at[idx_v, pl.ds(0, d - 1)],
                rows_v.at[:, pl.ds(0, d - 1)],
                sem,
            )
            pltpu.async_copy(lc_sh.at[idxhi_v], slab_v, sem2).wait()
            main.wait()
            for g in range(chunk // 16):
                iv = idx_v[pl.ds(g * 16, 16)]
                lo = lax.bitwise_and(iv, 127)
                rows16 = lax.iota(jnp.int32, 16) + g * 16
                vals = plsc.load_gather(slab_v, [rows16, lo])
                plsc.store_scatter(
                    rows_v, [rows16, jnp.full((16,), d - 1, jnp.int32)], vals
                )
            pltpu.sync_copy(rows_v, out_hbm.at[pl.ds(off, chunk)])
            return carry

        lax.fori_loop(0, n_chunks, chunk_body, 0, unroll=False)

    return k(flat_idx, table, lastcol)


def kernel(indices, embeddings):
    b, s = indices.shape
    v, d = embeddings.shape
    flat_idx = indices.reshape(b * s).astype(jnp.int32)
    lastcol = jnp.pad(
        embeddings[:, d - 1], (0, LC_ROWS * 128 - v)
    ).reshape(LC_ROWS, 128)
    out = _lookup(flat_idx, embeddings, lastcol, 128)
    return out.reshape(b, s, d)
